# baseline (speedup 1.0000x reference)
import functools

import jax
import jax.numpy as jnp
from jax import lax
from jax.experimental import pallas as pl
from jax.experimental.pallas import tpu as pltpu
from jax.experimental.pallas import tpu_sc as plsc

B = 16
C = 3
N = 4096
K = 8
CN = C * N
HALF = CN // 2
IHW = HALF + 16
P = 64
NCODE = K * K
TBL = 80
LANES = 16


def _sc_presence(x2, iexth, lev_pad):
    mesh = plsc.VectorSubcoreMesh(core_axis_name="c", subcore_axis_name="s")
    out_type = jax.ShapeDtypeStruct((2 * B, TBL), jnp.int32)

    @functools.partial(
        pl.kernel,
        mesh=mesh,
        out_type=out_type,
        compiler_params=pltpu.CompilerParams(
            needs_layout_passes=False, use_tc_tiling_on_sc=False),
        scratch_types=[
            pltpu.VMEM((2 * N,), jnp.float32),
            pltpu.VMEM((2 * N,), jnp.int32),
            pltpu.VMEM((IHW,), jnp.int32),
            pltpu.VMEM((128,), jnp.float32),
            pltpu.VMEM((TBL,), jnp.int32),
            pltpu.SemaphoreType.DMA,
            pltpu.SemaphoreType.DMA,
            pltpu.SemaphoreType.DMA,
        ],
    )
    def k(x_hbm, iexth_hbm, lev_hbm, pres_hbm,
          xb_v, sym_v, ih_v, lev_v, btab_v, sem0, sem1, sem2):
        wid = lax.axis_index("s") * 2 + lax.axis_index("c")
        half = (wid >= B).astype(jnp.int32)
        b = wid - half * B
        iota = lax.iota(jnp.int32, LANES)
        zero = iota * 0
        one = zero + 1

        cp_x = pltpu.async_copy(x_hbm.at[b, pl.ds(half * N, 2 * N)], xb_v,
                                sem0)
        cp_ih = pltpu.async_copy(iexth_hbm.at[half], ih_v, sem1)
        cp_lev = pltpu.async_copy(lev_hbm, lev_v, sem2)

        for t in range(TBL // LANES):
            btab_v[pl.ds(t * LANES, LANES)] = zero

        cp_lev.wait()
        cp_x.wait()
        for c_local in range(2):
            ch = half + c_local
            lsp = [plsc.load_gather(lev_v, [zero + (ch * K + m)])
                   for m in range(K)]

            @plsc.parallel_loop(0, N, step=LANES, unroll=4)
            def qbody(i, c_local=c_local, lsp=lsp):
                off = c_local * N + i
                v = xb_v[pl.ds(off, LANES)]
                ds_ = [jnp.abs(v - lsp[m]) for m in range(K)]
                idxs = [zero + m for m in range(K)]
                while len(ds_) > 1:
                    nd, ni = [], []
                    for t in range(0, len(ds_), 2):
                        take = ds_[t + 1] < ds_[t]
                        nd.append(jnp.where(take, ds_[t + 1], ds_[t]))
                        ni.append(jnp.where(take, idxs[t + 1], idxs[t]))
                    ds_, idxs = nd, ni
                sym_v[pl.ds(off, LANES)] = idxs[0]

        cp_ih.wait()

        @plsc.parallel_loop(0, HALF - LANES, step=LANES, unroll=4)
        def sbody(base):
            ia = ih_v[pl.ds(base, LANES)]
            ib = ih_v[pl.ds(base + 1, LANES)]
            ga = plsc.load_gather(sym_v, [ia])
            gb = plsc.load_gather(sym_v, [ib])
            plsc.store_scatter(btab_v, [ga * K + gb], one)

        base = HALF - LANES
        ia = ih_v[pl.ds(base, LANES)]
        ib = ih_v[pl.ds(base + 1, LANES)]
        ga = plsc.load_gather(sym_v, [ia])
        gb = plsc.load_gather(sym_v, [ib])
        code = ga * K + gb
        code = jnp.where((half == 0) | (iota < LANES - 1), code, NCODE)
        plsc.store_scatter(btab_v, [code], one)

        @pl.when(half == 1)
        def _stage():
            btab_v[pl.ds(NCODE, LANES)] = ga

        pltpu.async_copy(btab_v, pres_hbm.at[wid], sem0).wait()

    return k(x2, iexth, lev_pad)


def _tc_combine(pres, pmap_flat):
    def body(pres_ref, pmap_ref, out_ref):
        psh = pres_ref[...]
        psv = jnp.maximum(psh[0:B, 0:NCODE],
                          psh[B:2 * B, 0:NCODE]).astype(jnp.float32)
        sl = psh[B:2 * B, TBL - 1:TBL]

        pm = pmap_ref[...]
        nxt = jnp.concatenate([pm[:, 1:], pm[:, 0:1]], axis=1)
        codes = pm * K + nxt
        lane = lax.broadcasted_iota(jnp.int32, (P, N), 1)
        valid = lane < N - 1
        sh = codes & 31
        bit = lax.shift_left(jnp.ones((P, N), jnp.int32), sh)
        lo = jnp.where(valid & (codes < 32), bit, 0)
        hi = jnp.where(valid & (codes >= 32), bit, 0)
        w = N
        while w > 1:
            w //= 2
            lo = lo[:, 0:w] | lo[:, w:2 * w]
            hi = hi[:, 0:w] | hi[:, w:2 * w]
        kk = lax.broadcasted_iota(jnp.int32, (P, 32), 1)
        plo = lax.shift_right_logical(lo, kk) & 1
        phi = lax.shift_right_logical(hi, kk) & 1
        ppv = jnp.concatenate([plo, phi], axis=1).astype(jnp.float32)

        pf = pm[:, 0:1]
        cs = jnp.sum(psv, axis=1, keepdims=True)
        cp_col = jnp.sum(ppv, axis=1, keepdims=True)
        ones_b = jnp.ones((B, 1), jnp.float32)
        cp = lax.dot_general(ones_b, cp_col, (((1,), (1,)), ((), ())))
        inter = lax.dot_general(psv, ppv, (((1,), (1,)), ((), ())))

        el = (sl == lax.broadcasted_iota(jnp.int32, (B, K), 1))
        el = el.astype(jnp.float32)
        ef = (pf == lax.broadcasted_iota(jnp.int32, (P, K), 1))
        ef = ef.astype(jnp.float32)

        a_mat = el[:, 0:1] * psv[:, 0:K]
        for a in range(1, K):
            a_mat = a_mat + el[:, a:a + 1] * psv[:, a * K:(a + 1) * K]
        a_at = lax.dot_general(a_mat, ef, (((1,), (1,)), ((), ())))

        bp_cols = [jnp.sum(ef * ppv[:, l * K:(l + 1) * K], axis=1,
                           keepdims=True) for l in range(K)]
        bp = jnp.concatenate(bp_cols, axis=1)
        b_at = lax.dot_general(el, bp, (((1,), (1,)), ((), ())))

        uj = jnp.maximum(a_at, b_at)
        csp = cs + cp - inter + (1.0 - uj)
        cmin = jnp.minimum(cs, cp)
        cmax = jnp.maximum(cs, cp)
        out_ref[...] = (csp - cmin) / cmax

    return pl.pallas_call(
        body,
        out_shape=jax.ShapeDtypeStruct((B, P), jnp.float32),
    )(pres, pmap_flat)


def kernel(x, curve, levels, pmap):
    x2 = x.reshape(B, CN)
    curve = curve.astype(jnp.int32)
    ch_off = (jnp.arange(C, dtype=jnp.int32) * N)[:, None]
    idx = (curve[None, :] + ch_off).reshape(-1)
    row0 = idx[:IHW]
    row1 = jnp.concatenate([idx[HALF:], idx[-16:]]) - N
    iexth = jnp.stack([row0, row1])
    lev_pad = jnp.pad(levels.reshape(-1), (0, 128 - C * K))
    pmap_flat = pmap.reshape(P, N).astype(jnp.int32)

    pres = _sc_presence(x2, iexth, lev_pad)
    return _tc_combine(pres, pmap_flat)


# speedup vs baseline: 1.0670x; 1.0670x over previous
import functools

import jax
import jax.numpy as jnp
from jax import lax
from jax.experimental import pallas as pl
from jax.experimental.pallas import tpu as pltpu
from jax.experimental.pallas import tpu_sc as plsc

B = 16
C = 3
N = 4096
K = 8
CN = C * N
HALF = CN // 2
IHW = HALF + 16
P = 64
NCODE = K * K
TBL = 80
LANES = 16


def _sc_presence(x2, iexth, lev_pad):
    mesh = plsc.VectorSubcoreMesh(core_axis_name="c", subcore_axis_name="s")
    out_type = jax.ShapeDtypeStruct((2 * B, TBL), jnp.int32)

    @functools.partial(
        pl.kernel,
        mesh=mesh,
        out_type=out_type,
        compiler_params=pltpu.CompilerParams(
            needs_layout_passes=False, use_tc_tiling_on_sc=False),
        scratch_types=[
            pltpu.VMEM((2 * N,), jnp.float32),
            pltpu.VMEM((2 * N,), jnp.int32),
            pltpu.VMEM((IHW,), jnp.int32),
            pltpu.VMEM((128,), jnp.float32),
            pltpu.VMEM((TBL,), jnp.int32),
            pltpu.SemaphoreType.DMA,
            pltpu.SemaphoreType.DMA,
            pltpu.SemaphoreType.DMA,
        ],
    )
    def k(x_hbm, iexth_hbm, lev_hbm, pres_hbm,
          xb_v, sym_v, ih_v, lev_v, btab_v, sem0, sem1, sem2):
        wid = lax.axis_index("s") * 2 + lax.axis_index("c")
        half = (wid >= B).astype(jnp.int32)
        b = wid - half * B
        iota = lax.iota(jnp.int32, LANES)
        zero = iota * 0
        one = zero + 1

        cp_x = pltpu.async_copy(x_hbm.at[b, pl.ds(half * N, 2 * N)], xb_v,
                                sem0)
        cp_ih = pltpu.async_copy(iexth_hbm.at[half], ih_v, sem1)
        cp_lev = pltpu.async_copy(lev_hbm, lev_v, sem2)

        for t in range(TBL // LANES):
            btab_v[pl.ds(t * LANES, LANES)] = zero

        cp_lev.wait()
        cp_x.wait()
        for c_local in range(2):
            ch = half + c_local
            lsp = [plsc.load_gather(lev_v, [zero + (ch * K + m)])
                   for m in range(K)]

            @plsc.parallel_loop(0, N, step=LANES, unroll=4)
            def qbody(i, c_local=c_local, lsp=lsp):
                off = c_local * N + i
                v = xb_v[pl.ds(off, LANES)]
                ds_ = [jnp.abs(v - lsp[m]) for m in range(K)]
                idxs = [zero + m for m in range(K)]
                while len(ds_) > 1:
                    nd, ni = [], []
                    for t in range(0, len(ds_), 2):
                        take = ds_[t + 1] < ds_[t]
                        nd.append(jnp.where(take, ds_[t + 1], ds_[t]))
                        ni.append(jnp.where(take, idxs[t + 1], idxs[t]))
                    ds_, idxs = nd, ni
                sym_v[pl.ds(off, LANES)] = idxs[0]

        cp_ih.wait()

        @plsc.parallel_loop(0, HALF - LANES, step=LANES, unroll=4)
        def sbody(base):
            ia = ih_v[pl.ds(base, LANES)]
            ib = ih_v[pl.ds(base + 1, LANES)]
            ga = plsc.load_gather(sym_v, [ia])
            gb = plsc.load_gather(sym_v, [ib])
            plsc.store_scatter(btab_v, [ga * K + gb], one)

        base = HALF - LANES
        ia = ih_v[pl.ds(base, LANES)]
        ib = ih_v[pl.ds(base + 1, LANES)]
        ga = plsc.load_gather(sym_v, [ia])
        gb = plsc.load_gather(sym_v, [ib])
        code = ga * K + gb
        code = jnp.where((half == 0) | (iota < LANES - 1), code, NCODE)
        plsc.store_scatter(btab_v, [code], one)

        @pl.when(half == 1)
        def _stage():
            btab_v[pl.ds(NCODE, LANES)] = ga

        pltpu.async_copy(btab_v, pres_hbm.at[wid], sem0).wait()

    return k(x2, iexth, lev_pad)


def _tc_proto(pmap_flat):
    def body(pmap_ref, out_ref):
        pm = pmap_ref[...]
        nxt = jnp.concatenate([pm[:, 1:], pm[:, 0:1]], axis=1)
        codes = pm * K + nxt
        lane = lax.broadcasted_iota(jnp.int32, (P, N), 1)
        valid = lane < N - 1
        sh = codes & 31
        bit = lax.shift_left(jnp.ones((P, N), jnp.int32), sh)
        lo = jnp.where(valid & (codes < 32), bit, 0)
        hi = jnp.where(valid & (codes >= 32), bit, 0)
        w = N
        while w > 1:
            w //= 2
            lo = lo[:, 0:w] | lo[:, w:2 * w]
            hi = hi[:, 0:w] | hi[:, w:2 * w]
        kk = lax.broadcasted_iota(jnp.int32, (P, 32), 1)
        plo = lax.shift_right_logical(lo, kk) & 1
        phi = lax.shift_right_logical(hi, kk) & 1
        out_ref[...] = jnp.concatenate([plo, phi], axis=1).astype(jnp.float32)

    return pl.pallas_call(
        body,
        out_shape=jax.ShapeDtypeStruct((P, NCODE), jnp.float32),
    )(pmap_flat)


def _tc_combine(pres, ppv_in, pf_in):
    def body(pres_ref, ppv_ref, pf_ref, out_ref):
        psh = pres_ref[...]
        psv = jnp.maximum(psh[0:B, 0:NCODE],
                          psh[B:2 * B, 0:NCODE]).astype(jnp.float32)
        sl = psh[B:2 * B, TBL - 1:TBL]
        ppv = ppv_ref[...]
        pf = pf_ref[...]
        cs = jnp.sum(psv, axis=1, keepdims=True)
        cp_col = jnp.sum(ppv, axis=1, keepdims=True)
        ones_b = jnp.ones((B, 1), jnp.float32)
        cp = lax.dot_general(ones_b, cp_col, (((1,), (1,)), ((), ())))
        inter = lax.dot_general(psv, ppv, (((1,), (1,)), ((), ())))

        el = (sl == lax.broadcasted_iota(jnp.int32, (B, K), 1))
        el = el.astype(jnp.float32)
        ef = (pf == lax.broadcasted_iota(jnp.int32, (P, K), 1))
        ef = ef.astype(jnp.float32)

        a_mat = el[:, 0:1] * psv[:, 0:K]
        for a in range(1, K):
            a_mat = a_mat + el[:, a:a + 1] * psv[:, a * K:(a + 1) * K]
        a_at = lax.dot_general(a_mat, ef, (((1,), (1,)), ((), ())))

        bp_cols = [jnp.sum(ef * ppv[:, l * K:(l + 1) * K], axis=1,
                           keepdims=True) for l in range(K)]
        bp = jnp.concatenate(bp_cols, axis=1)
        b_at = lax.dot_general(el, bp, (((1,), (1,)), ((), ())))

        uj = jnp.maximum(a_at, b_at)
        csp = cs + cp - inter + (1.0 - uj)
        cmin = jnp.minimum(cs, cp)
        cmax = jnp.maximum(cs, cp)
        out_ref[...] = (csp - cmin) / cmax

    return pl.pallas_call(
        body,
        out_shape=jax.ShapeDtypeStruct((B, P), jnp.float32),
    )(pres, ppv_in, pf_in)


def kernel(x, curve, levels, pmap):
    x2 = x.reshape(B, CN)
    curve = curve.astype(jnp.int32)
    ch_off = (jnp.arange(C, dtype=jnp.int32) * N)[:, None]
    idx = (curve[None, :] + ch_off).reshape(-1)
    row0 = idx[:IHW]
    row1 = jnp.concatenate([idx[HALF:], idx[-16:]]) - N
    iexth = jnp.stack([row0, row1])
    lev_pad = jnp.pad(levels.reshape(-1), (0, 128 - C * K))
    pmap_flat = pmap.reshape(P, N).astype(jnp.int32)

    ppv = _tc_proto(pmap_flat)
    pres = _sc_presence(x2, iexth, lev_pad)
    pf = pmap_flat[:, 0:1]
    return _tc_combine(pres, ppv, pf)
